# baseline (speedup 1.0000x reference)
import jax
import jax.numpy as jnp
from jax import lax
from jax.experimental import pallas as pl
from jax.experimental.pallas import tpu as pltpu
from jax.experimental.pallas import tpu_sc as plsc

B = 16384
D = 16
NC = 2
NS = 16
NW = NC * NS
BPW = B // NW
CH = 128
NCH = BPW // CH
L = 16
GPC = CH // L
ROWS_PER_BLOCK = 128 // D


def _sc_body(bu_hbm, bp_hbm, bn_hbm, ut_hbm, it_hbm,
             outp_hbm, outn_hbm,
             idx_u, idx_p, idx_n,
             id8_u, id8_p, id8_n,
             sub_u, sub_p, sub_n,
             rows_u, rows_i, rows_j,
             accp, accn, sem_a, sem_b):
  wid = lax.axis_index("s") * NC + lax.axis_index("c")

  pltpu.sync_copy(bu_hbm.at[wid], idx_u)
  pltpu.sync_copy(bp_hbm.at[wid], idx_p)
  pltpu.sync_copy(bn_hbm.at[wid], idx_n)

  for raw, id8, sub in ((idx_u, id8_u, sub_u),
                        (idx_p, id8_p, sub_p),
                        (idx_n, id8_n, sub_n)):
    for c in range(NCH):
      for k in range(CH // L):
        s = pl.ds(k * L, L)
        v = raw[c, s]
        id8[c, s] = lax.shift_right_logical(v, 3)
        sub[c, s] = (v & 7) * D

  sems = (sem_a, sem_b)

  def fire(c):
    buf = c % 2
    cps = (pltpu.make_async_copy(ut_hbm.at[id8_u.at[c]], rows_u.at[buf],
                                 sems[buf]),
           pltpu.make_async_copy(it_hbm.at[id8_p.at[c]], rows_i.at[buf],
                                 sems[buf]),
           pltpu.make_async_copy(it_hbm.at[id8_n.at[c]], rows_j.at[buf],
                                 sems[buf]))
    for cp in cps:
      cp.start()
    return cps

  lane = lax.iota(jnp.int32, L)
  inflight = {0: fire(0)}

  for c in range(NCH):
    if c + 1 < NCH:
      inflight[c + 1] = fire(c + 1)
    for cp in inflight.pop(c):
      cp.wait()

    buf = c % 2
    ru, ri, rj = rows_u.at[buf], rows_i.at[buf], rows_j.at[buf]

    def group(g, carry):
      rows = g * L + lane
      su = sub_u[c, pl.ds(g * L, L)]
      sp = sub_p[c, pl.ds(g * L, L)]
      sn = sub_n[c, pl.ds(g * L, L)]
      ap = jnp.zeros((L,), jnp.float32)
      an = jnp.zeros((L,), jnp.float32)
      for d in range(D):
        uu = plsc.load_gather(ru, [rows, su + d])
        ii = plsc.load_gather(ri, [rows, sp + d])
        jj = plsc.load_gather(rj, [rows, sn + d])
        ap = ap + uu * ii
        an = an + uu * jj
      base = c * CH + pl.multiple_of(g * L, L)
      accp[pl.ds(base, L)] = ap
      accn[pl.ds(base, L)] = an
      return carry

    lax.fori_loop(0, GPC, group, 0)

  out = pl.ds(wid * BPW, BPW)
  pltpu.sync_copy(accp, outp_hbm.at[out])
  pltpu.sync_copy(accn, outn_hbm.at[out])


@jax.jit
def kernel(batch_user, batch_pos_item, batch_neg_item, user_table, item_table):
  bu = batch_user.reshape(NW, NCH, CH)
  bp = batch_pos_item.reshape(NW, NCH, CH)
  bn = batch_neg_item.reshape(NW, NCH, CH)
  ut = user_table.reshape(-1, 128)
  it = item_table.reshape(-1, 128)

  mesh = plsc.VectorSubcoreMesh(core_axis_name="c", subcore_axis_name="s",
                                num_cores=NC, num_subcores=NS)
  run = pl.kernel(
      _sc_body,
      out_type=(jax.ShapeDtypeStruct((B,), jnp.float32),
                jax.ShapeDtypeStruct((B,), jnp.float32)),
      mesh=mesh,
      scratch_types=[
          pltpu.VMEM((NCH, CH), jnp.int32),
          pltpu.VMEM((NCH, CH), jnp.int32),
          pltpu.VMEM((NCH, CH), jnp.int32),
          pltpu.VMEM((NCH, CH), jnp.int32),
          pltpu.VMEM((NCH, CH), jnp.int32),
          pltpu.VMEM((NCH, CH), jnp.int32),
          pltpu.VMEM((NCH, CH), jnp.int32),
          pltpu.VMEM((NCH, CH), jnp.int32),
          pltpu.VMEM((NCH, CH), jnp.int32),
          pltpu.VMEM((2, CH, 128), jnp.float32),
          pltpu.VMEM((2, CH, 128), jnp.float32),
          pltpu.VMEM((2, CH, 128), jnp.float32),
          pltpu.VMEM((BPW,), jnp.float32),
          pltpu.VMEM((BPW,), jnp.float32),
          pltpu.SemaphoreType.DMA,
          pltpu.SemaphoreType.DMA,
      ],
      compiler_params=pltpu.CompilerParams(needs_layout_passes=False),
  )
  pos, neg = run(bu, bp, bn, ut, it)
  return (pos.reshape(B, 1), neg.reshape(B, 1))


# speedup vs baseline: 4.6669x; 4.6669x over previous
import jax
import jax.numpy as jnp
from jax import lax
from jax.experimental import pallas as pl
from jax.experimental.pallas import tpu as pltpu
from jax.experimental.pallas import tpu_sc as plsc

B = 16384
D = 16
NC = 2
NS = 16
NW = NC * NS
BPW = B // NW
L = 16
NG = BPW // L
BLK = 128


def _sc_body(bu_hbm, bp_hbm, bn_hbm, ut_hbm, it_hbm,
             outp_hbm, outn_hbm,
             vidx_u, vidx_p, vidx_n,
             blk_u, blk_i, blk_j,
             accp, accn, sem):
  wid = lax.axis_index("s") * NC + lax.axis_index("c")

  pltpu.sync_copy(bu_hbm.at[wid], vidx_u)
  pltpu.sync_copy(bp_hbm.at[wid], vidx_p)
  pltpu.sync_copy(bn_hbm.at[wid], vidx_n)

  lane = lax.iota(jnp.int32, L)

  def group(g, carry):
    row = g // 8
    col0 = (g % 8) * L
    vu = vidx_u[row, pl.ds(col0, L)]
    vp = vidx_p[row, pl.ds(col0, L)]
    vn = vidx_n[row, pl.ds(col0, L)]
    su = lax.shift_right_logical(vu, 7) * BLK
    sp = lax.shift_right_logical(vp, 7) * BLK
    sn = lax.shift_right_logical(vn, 7) * BLK
    copies = []
    for r in range(L):
      for starts, tbl, dst in ((su, ut_hbm, blk_u),
                               (sp, it_hbm, blk_i),
                               (sn, it_hbm, blk_j)):
        start = pl.multiple_of(starts[r], BLK)
        copies.append(pltpu.make_async_copy(
            tbl.at[:, pl.ds(start, BLK)], dst.at[r], sem))
    for cp in copies:
      cp.start()
    for cp in copies:
      cp.wait()

    ru = vu & (BLK - 1)
    rp = vp & (BLK - 1)
    rn = vn & (BLK - 1)
    ap = jnp.zeros((L,), jnp.float32)
    an = jnp.zeros((L,), jnp.float32)
    for c in range(D):
      cv = jnp.full((L,), c, jnp.int32)
      uu = plsc.load_gather(blk_u, [lane, cv, ru])
      ii = plsc.load_gather(blk_i, [lane, cv, rp])
      jj = plsc.load_gather(blk_j, [lane, cv, rn])
      ap = ap + uu * ii
      an = an + uu * jj
    base = pl.multiple_of(g * L, L)
    accp[pl.ds(base, L)] = ap
    accn[pl.ds(base, L)] = an
    return carry

  lax.fori_loop(0, NG, group, 0)

  out = pl.ds(wid * BPW, BPW)
  pltpu.sync_copy(accp, outp_hbm.at[out])
  pltpu.sync_copy(accn, outn_hbm.at[out])


@jax.jit
def kernel(batch_user, batch_pos_item, batch_neg_item, user_table, item_table):
  bu = batch_user.reshape(NW, 4, 128)
  bp = batch_pos_item.reshape(NW, 4, 128)
  bn = batch_neg_item.reshape(NW, 4, 128)
  ut = user_table.T
  it = item_table.T

  mesh = plsc.VectorSubcoreMesh(core_axis_name="c", subcore_axis_name="s",
                                num_cores=NC, num_subcores=NS)
  run = pl.kernel(
      _sc_body,
      out_type=(jax.ShapeDtypeStruct((B,), jnp.float32),
                jax.ShapeDtypeStruct((B,), jnp.float32)),
      mesh=mesh,
      scratch_types=[
          pltpu.VMEM((4, 128), jnp.int32),
          pltpu.VMEM((4, 128), jnp.int32),
          pltpu.VMEM((4, 128), jnp.int32),
          pltpu.VMEM((L, D, BLK), jnp.float32),
          pltpu.VMEM((L, D, BLK), jnp.float32),
          pltpu.VMEM((L, D, BLK), jnp.float32),
          pltpu.VMEM((BPW,), jnp.float32),
          pltpu.VMEM((BPW,), jnp.float32),
          pltpu.SemaphoreType.DMA,
      ],
      compiler_params=pltpu.CompilerParams(needs_layout_passes=False,
                                           disable_bounds_checks=True),
  )
  pos, neg = run(bu, bp, bn, ut, it)
  return (pos.reshape(B, 1), neg.reshape(B, 1))
